# Initial kernel scaffold; baseline (speedup 1.0000x reference)
import jax
import jax.numpy as jnp
from jax.experimental import pallas as pl


def kernel(edge_index, n, num_sample, k, edge_weights):
    raise NotImplementedError("write your pallas kernel here")


# speedup vs baseline: 1.2611x; 1.2611x over previous
import jax
import jax.numpy as jnp
from jax.experimental import pallas as pl
from jax.experimental.pallas import tpu as pltpu


def _logp_kernel(k_ref, ew_ref, out_ref):
    k = k_ref[0]
    row = ew_ref[pl.ds(k, 1), :]
    m = jnp.max(row)
    e = jnp.exp(row - m)
    s = jnp.sum(e)
    p = e / s
    out_ref[0] = jnp.sum(jnp.log(p + 1e-8))


def kernel(edge_index, n, num_sample, k, edge_weights):
    k_arr = jnp.reshape(jnp.asarray(k, jnp.int32), (1,))
    log_p = pl.pallas_call(
        _logp_kernel,
        out_shape=jax.ShapeDtypeStruct((1,), jnp.float32),
        in_specs=[
            pl.BlockSpec(memory_space=pltpu.SMEM),
            pl.BlockSpec(memory_space=pltpu.VMEM),
        ],
        out_specs=pl.BlockSpec(memory_space=pltpu.SMEM),
    )(k_arr, edge_weights)
    return (edge_index, log_p[0])
